# baseline (speedup 1.0000x reference)
import jax
import jax.numpy as jnp
from jax.experimental import pallas as pl
from jax.experimental.pallas import tpu as pltpu

B = 512
L = 128
F = 50
D = 64
H = 32
E = 8
R = 2 * B
VOCAB = 901
HEAD = 32
BT = 128
TC = 16
NCH = L // TC
NEG = -3.0e38


def _stage1_kernel(hor_ref, W_in_ref, emb_ref, W_gate_ref, b_gate_ref,
                   Wih0T_ref, he_ref, eid_ref, ws_ref, M_ref):
    i = pl.program_id(0)

    hor = hor_ref[...].astype(jnp.int32)
    iota_v = jax.lax.broadcasted_iota(jnp.int32, (BT, VOCAB), 1)
    oh = (hor[:, None] == iota_v).astype(jnp.float32)
    he = jnp.dot(oh, emb_ref[...],
                 preferred_element_type=jnp.float32)
    he_ref[...] = he

    logits = jax.lax.dot_general(he, W_gate_ref[...], (((1,), (1,)), ((), ())),
                                 preferred_element_type=jnp.float32)
    logits = logits + b_gate_ref[...][None, :]
    iota_e = jax.lax.broadcasted_iota(jnp.int32, (BT, E), 1)
    m1 = jnp.max(logits, axis=1, keepdims=True)
    is1 = (logits == m1)
    idx1 = jnp.min(jnp.where(is1, iota_e, E), axis=1, keepdims=True)
    masked = jnp.where(iota_e == idx1, NEG, logits)
    m2 = jnp.max(masked, axis=1, keepdims=True)
    is2 = (masked == m2)
    idx2 = jnp.min(jnp.where(is2, iota_e, E), axis=1, keepdims=True)
    w1 = 1.0 / (1.0 + jnp.exp(m2 - m1))
    w2 = 1.0 - w1
    eid_ref[...] = jnp.concatenate([idx1[None], idx2[None]], axis=0)
    ws_ref[...] = jnp.concatenate([w1[None], w2[None]], axis=0)

    @pl.when(i == 0)
    def _():
        W_in = W_in_ref[...]
        for e in range(E):
            M_ref[e] = jax.lax.dot_general(
                W_in, Wih0T_ref[e], (((0,), (0,)), ((), ())),
                preferred_element_type=jnp.float32)


def _tile8(v):
    return jnp.concatenate([v] * 8, axis=1)


def _stage2_kernel(x_ref, he_ref, eid_ref, ws_ref, binr_ref, M_ref,
                   Wih0v_ref, WA_ref, Whh1b_ref, bih0_ref, bhh0_ref, b1_ref,
                   Wh1s_ref, bh1_ref, Wh2_ref, bh2_ref, out_ref,
                   h0_ref, h1_ref, m0_ref, g0c_ref, b0h_ref, b1r_ref,
                   bh1r_ref, wh2r_ref, bh2r_ref):
    c = pl.program_id(0)
    eidc = eid_ref[...]
    oh_e = (jax.lax.broadcasted_iota(jnp.int32, (R, E), 1)
            == eidc).astype(jnp.float32)

    @pl.when(c == 0)
    def _():
        m0 = (jax.lax.broadcasted_iota(jnp.int32, (R, E * H), 1) // H
              == eidc).astype(jnp.float32)
        m64 = (jax.lax.broadcasted_iota(jnp.int32, (R, E * D), 1) // D
               == eidc).astype(jnp.float32)
        m0_ref[...] = m0
        heR = jnp.concatenate([he_ref[...], he_ref[...]], axis=0)
        hb = (heR + binr_ref[...])
        hec = jnp.dot(_tile8(hb) * m64, Wih0v_ref[...],
                      preferred_element_type=jnp.float32)
        g0c_ref[...] = hec + jnp.dot(oh_e, bih0_ref[...],
                                     preferred_element_type=jnp.float32)
        b0h_ref[...] = jnp.dot(oh_e, bhh0_ref[...],
                               preferred_element_type=jnp.float32)
        b1r_ref[...] = jnp.dot(oh_e, b1_ref[...],
                               preferred_element_type=jnp.float32)
        bh1r_ref[...] = jnp.dot(oh_e, bh1_ref[...],
                                preferred_element_type=jnp.float32)
        wh2r_ref[...] = jnp.dot(oh_e, Wh2_ref[...],
                                preferred_element_type=jnp.float32)
        bh2r_ref[...] = jnp.dot(oh_e, bh2_ref[...],
                                preferred_element_type=jnp.float32)
        h0_ref[...] = jnp.zeros((R, H), jnp.float32)
        h1_ref[...] = jnp.zeros((R, H), jnp.float32)

    m0 = m0_ref[...]
    g0c = g0c_ref[...]
    b0h = b0h_ref[...]
    b1r = b1r_ref[...]
    h0 = h0_ref[...]
    h1 = h1_ref[...]
    WA = WA_ref[...]
    Whh1b = Whh1b_ref[...]

    sel0 = [oh_e[:B, e].reshape(B, 1, 1) for e in range(E)]
    sel1 = [oh_e[B:, e].reshape(B, 1, 1) for e in range(E)]

    xf = x_ref[...].reshape(B * TC, F)
    gs0 = jnp.zeros((B, TC, 3 * H), jnp.float32)
    gs1 = jnp.zeros((B, TC, 3 * H), jnp.float32)
    for e in range(E):
        ge = jnp.dot(xf, M_ref[e], preferred_element_type=jnp.float32)
        ge = ge.reshape(B, TC, 3 * H)
        gs0 = gs0 + ge * sel0[e]
        gs1 = gs1 + ge * sel1[e]
    gs0 = gs0 + g0c[:B][:, None, :]
    gs1 = gs1 + g0c[B:][:, None, :]

    for i in range(TC):
        gi0 = jnp.concatenate([gs0[:, i, :], gs1[:, i, :]], axis=0)
        A = _tile8(h0) * m0
        GA = jnp.dot(A, WA, preferred_element_type=jnp.float32)
        Bt = _tile8(h1) * m0
        g1b = jnp.dot(Bt, Whh1b, preferred_element_type=jnp.float32)
        gh0 = GA[:, :3 * H] + b0h
        g1 = GA[:, 3 * H:] + g1b + b1r

        r = jax.nn.sigmoid(gi0[:, :H] + gh0[:, :H])
        z = jax.nn.sigmoid(gi0[:, H:2 * H] + gh0[:, H:2 * H])
        n = jnp.tanh(gi0[:, 2 * H:] + r * gh0[:, 2 * H:])
        h0 = (1.0 - z) * n + z * h0

        r1 = jax.nn.sigmoid(g1[:, :H])
        z1 = jax.nn.sigmoid(g1[:, H:2 * H])
        n1 = jnp.tanh(g1[:, 2 * H:3 * H] + r1 * g1[:, 3 * H:])
        h1n = (1.0 - z1) * n1 + z1 * h1
        if i == 0:
            h1 = h1n * (c > 0).astype(jnp.float32)
        else:
            h1 = h1n

    h0_ref[...] = h0
    h1_ref[...] = h1

    @pl.when(c == NCH - 1)
    def _():
        A = _tile8(h0) * m0
        g1a = jnp.dot(A, WA, preferred_element_type=jnp.float32)[:, 3 * H:]
        Bt = _tile8(h1) * m0
        g1b = jnp.dot(Bt, Whh1b, preferred_element_type=jnp.float32)
        g1 = g1a + g1b + b1r
        r1 = jax.nn.sigmoid(g1[:, :H])
        z1 = jax.nn.sigmoid(g1[:, H:2 * H])
        n1 = jnp.tanh(g1[:, 2 * H:3 * H] + r1 * g1[:, 3 * H:])
        h1f = (1.0 - z1) * n1 + z1 * h1

        zh = jnp.dot(_tile8(h1f) * m0, Wh1s_ref[...],
                     preferred_element_type=jnp.float32) + bh1r_ref[...]
        zh_r = jnp.maximum(zh, 0.0)
        pred = jnp.sum(zh_r * wh2r_ref[...], axis=1, keepdims=True)
        pred = pred + bh2r_ref[...]
        ws = ws_ref[...]
        out_ref[...] = (ws[:B] * pred[:B]) + (ws[B:] * pred[B:])


@jax.jit
def kernel(x, horizon, W_in, b_in, emb, W_gate, b_gate, W_ih0, W_hh0, b_ih0,
           b_hh0, W_ih1, W_hh1, b_ih1, b_hh1, W_h1, b_h1, W_h2, b_h2):
    x = x.astype(jnp.float32)
    horizon = horizon.astype(jnp.int32)
    Wih0T = W_ih0.transpose(0, 2, 1)

    he, eid, ws, M = pl.pallas_call(
        _stage1_kernel,
        grid=(B // BT,),
        in_specs=[
            pl.BlockSpec((BT,), lambda i: (i,)),
            pl.BlockSpec((D, F), lambda i: (0, 0)),
            pl.BlockSpec((VOCAB, D), lambda i: (0, 0)),
            pl.BlockSpec((E, D), lambda i: (0, 0)),
            pl.BlockSpec((E,), lambda i: (0,)),
            pl.BlockSpec((E, D, 3 * H), lambda i: (0, 0, 0)),
        ],
        out_specs=[
            pl.BlockSpec((BT, D), lambda i: (i, 0)),
            pl.BlockSpec((2, BT, 1), lambda i: (0, i, 0)),
            pl.BlockSpec((2, BT, 1), lambda i: (0, i, 0)),
            pl.BlockSpec((E, F, 3 * H), lambda i: (0, 0, 0)),
        ],
        out_shape=[
            jax.ShapeDtypeStruct((B, D), jnp.float32),
            jax.ShapeDtypeStruct((2, B, 1), jnp.int32),
            jax.ShapeDtypeStruct((2, B, 1), jnp.float32),
            jax.ShapeDtypeStruct((E, F, 3 * H), jnp.float32),
        ],
    )(horizon, W_in, emb, W_gate, b_gate, Wih0T)

    Wih0v = Wih0T.reshape(E * D, 3 * H)
    Whh0s = W_hh0.transpose(0, 2, 1).reshape(E * H, 3 * H)
    Wih1T = W_ih1.transpose(0, 2, 1)
    Whh1T = W_hh1.transpose(0, 2, 1)
    zH = jnp.zeros((E, H, H), jnp.float32)
    Wih1s = jnp.concatenate([Wih1T[:, :, :2 * H], Wih1T[:, :, 2 * H:], zH],
                            2).reshape(E * H, 4 * H)
    Whh1b = jnp.concatenate([Whh1T[:, :, :2 * H], zH, Whh1T[:, :, 2 * H:]],
                            2).reshape(E * H, 4 * H)
    WA = jnp.concatenate([Whh0s, Wih1s], axis=1)
    b1 = jnp.concatenate([b_ih1[:, :2 * H] + b_hh1[:, :2 * H],
                          b_ih1[:, 2 * H:], b_hh1[:, 2 * H:]], axis=1)
    Wh1s = W_h1.transpose(0, 2, 1).reshape(E * H, HEAD)

    out = pl.pallas_call(
        _stage2_kernel,
        grid=(NCH,),
        in_specs=[
            pl.BlockSpec((B, TC, F), lambda c: (0, c, 0)),
            pl.BlockSpec((B, D), lambda c: (0, 0)),
            pl.BlockSpec((R, 1), lambda c: (0, 0)),
            pl.BlockSpec((R, 1), lambda c: (0, 0)),
            pl.BlockSpec((1, D), lambda c: (0, 0)),
            pl.BlockSpec((E, F, 3 * H), lambda c: (0, 0, 0)),
            pl.BlockSpec((E * D, 3 * H), lambda c: (0, 0)),
            pl.BlockSpec((E * H, 7 * H), lambda c: (0, 0)),
            pl.BlockSpec((E * H, 4 * H), lambda c: (0, 0)),
            pl.BlockSpec((E, 3 * H), lambda c: (0, 0)),
            pl.BlockSpec((E, 3 * H), lambda c: (0, 0)),
            pl.BlockSpec((E, 4 * H), lambda c: (0, 0)),
            pl.BlockSpec((E * H, HEAD), lambda c: (0, 0)),
            pl.BlockSpec((E, HEAD), lambda c: (0, 0)),
            pl.BlockSpec((E, HEAD), lambda c: (0, 0)),
            pl.BlockSpec((E, 1), lambda c: (0, 0)),
        ],
        out_specs=pl.BlockSpec((B, 1), lambda c: (0, 0)),
        out_shape=jax.ShapeDtypeStruct((B, 1), jnp.float32),
        scratch_shapes=[pltpu.VMEM((R, H), jnp.float32),
                        pltpu.VMEM((R, H), jnp.float32),
                        pltpu.VMEM((R, E * H), jnp.float32),
                        pltpu.VMEM((R, 3 * H), jnp.float32),
                        pltpu.VMEM((R, 3 * H), jnp.float32),
                        pltpu.VMEM((R, 4 * H), jnp.float32),
                        pltpu.VMEM((R, HEAD), jnp.float32),
                        pltpu.VMEM((R, HEAD), jnp.float32),
                        pltpu.VMEM((R, 1), jnp.float32)],
    )(x, he, eid.reshape(R, 1), ws.reshape(R, 1), b_in.reshape(1, D), M,
      Wih0v, WA, Whh1b, b_ih0, b_hh0, b1, Wh1s, b_h1, W_h2.reshape(E, HEAD),
      b_h2)

    return out[:, 0]


# speedup vs baseline: 1.0702x; 1.0702x over previous
import jax
import jax.numpy as jnp
from jax.experimental import pallas as pl
from jax.experimental.pallas import tpu as pltpu

B = 512
L = 128
F = 50
D = 64
H = 32
E = 8
R = 2 * B
VOCAB = 901
HEAD = 32
BT = 128
TC = 16
NCH = L // TC
NEG = -3.0e38


def _stage1_kernel(x_ref, hor_ref, W_in_ref, b_in_ref, emb_ref, W_gate_ref,
                   b_gate_ref, xp_ref, eid_ref, ws_ref):
    hor = hor_ref[...].astype(jnp.int32)
    iota_v = jax.lax.broadcasted_iota(jnp.int32, (BT, VOCAB), 1)
    oh = (hor[:, None] == iota_v).astype(jnp.float32)
    he = jnp.dot(oh, emb_ref[...],
                 preferred_element_type=jnp.float32)

    xt = x_ref[...].reshape(BT * L, F)
    xp = jax.lax.dot_general(xt, W_in_ref[...], (((1,), (1,)), ((), ())),
                             preferred_element_type=jnp.float32)
    xp = xp + b_in_ref[...][None, :]
    xp = xp.reshape(BT, L, D) + he[:, None, :]
    xp_ref[...] = xp

    logits = jax.lax.dot_general(he, W_gate_ref[...], (((1,), (1,)), ((), ())),
                                 preferred_element_type=jnp.float32)
    logits = logits + b_gate_ref[...][None, :]
    iota_e = jax.lax.broadcasted_iota(jnp.int32, (BT, E), 1)
    m1 = jnp.max(logits, axis=1, keepdims=True)
    is1 = (logits == m1)
    idx1 = jnp.min(jnp.where(is1, iota_e, E), axis=1, keepdims=True)
    masked = jnp.where(iota_e == idx1, NEG, logits)
    m2 = jnp.max(masked, axis=1, keepdims=True)
    is2 = (masked == m2)
    idx2 = jnp.min(jnp.where(is2, iota_e, E), axis=1, keepdims=True)
    w1 = 1.0 / (1.0 + jnp.exp(m2 - m1))
    w2 = 1.0 - w1
    eid_ref[...] = jnp.concatenate([idx1[None], idx2[None]], axis=0)
    ws_ref[...] = jnp.concatenate([w1[None], w2[None]], axis=0)


def _tile8(v):
    return jnp.concatenate([v] * 8, axis=1)


def _stage2_kernel(xp_ref, eid_ref, ws_ref, Wih0T_ref, Whh0s_ref, bih0_ref,
                   bhh0_ref, W1s_ref, b1_ref, Wh1s_ref, bh1_ref, Wh2_ref,
                   bh2_ref, out_ref, h0_ref, h1_ref):
    c = pl.program_id(0)
    eidc = eid_ref[...]
    oh_e = (jax.lax.broadcasted_iota(jnp.int32, (R, E), 1)
            == eidc).astype(jnp.float32)
    m0 = (jax.lax.broadcasted_iota(jnp.int32, (R, E * H), 1) // H
          == eidc).astype(jnp.float32)
    m1 = (jax.lax.broadcasted_iota(jnp.int32, (R, E * 2 * H), 1) // (2 * H)
          == eidc).astype(jnp.float32)

    b0i = jnp.dot(oh_e, bih0_ref[...], preferred_element_type=jnp.float32)
    b0h = jnp.dot(oh_e, bhh0_ref[...], preferred_element_type=jnp.float32)
    b1r = jnp.dot(oh_e, b1_ref[...], preferred_element_type=jnp.float32)
    bh1r = jnp.dot(oh_e, bh1_ref[...], preferred_element_type=jnp.float32)
    wh2r = jnp.dot(oh_e, Wh2_ref[...], preferred_element_type=jnp.float32)
    bh2r = jnp.dot(oh_e, bh2_ref[...], preferred_element_type=jnp.float32)

    sel0 = [oh_e[:B, e].reshape(B, 1, 1) for e in range(E)]
    sel1 = [oh_e[B:, e].reshape(B, 1, 1) for e in range(E)]

    Whh0s = Whh0s_ref[...]
    W1s = W1s_ref[...]

    @pl.when(c == 0)
    def _():
        h0_ref[...] = jnp.zeros((R, H), jnp.float32)
        h1_ref[...] = jnp.zeros((R, H), jnp.float32)

    h0 = h0_ref[...]
    h1 = h1_ref[...]

    xf = xp_ref[...].reshape(B * TC, D)
    gs0 = jnp.zeros((B, TC, 3 * H), jnp.float32)
    gs1 = jnp.zeros((B, TC, 3 * H), jnp.float32)
    for e in range(E):
        ge = jnp.dot(xf, Wih0T_ref[e],
                     preferred_element_type=jnp.float32)
        ge = ge.reshape(B, TC, 3 * H)
        gs0 = gs0 + ge * sel0[e]
        gs1 = gs1 + ge * sel1[e]
    gs0 = gs0 + b0i[:B][:, None, :]
    gs1 = gs1 + b0i[B:][:, None, :]

    for i in range(TC):
        gi0 = jnp.concatenate([gs0[:, i, :], gs1[:, i, :]], axis=0)
        gh0 = jnp.dot(_tile8(h0) * m0, Whh0s,
                      preferred_element_type=jnp.float32) + b0h
        cat1 = jnp.concatenate([h0, h1], axis=1)
        g1 = jnp.dot(_tile8(cat1) * m1, W1s,
                     preferred_element_type=jnp.float32) + b1r

        r = jax.nn.sigmoid(gi0[:, :H] + gh0[:, :H])
        z = jax.nn.sigmoid(gi0[:, H:2 * H] + gh0[:, H:2 * H])
        n = jnp.tanh(gi0[:, 2 * H:] + r * gh0[:, 2 * H:])
        h0 = (1.0 - z) * n + z * h0

        r1 = jax.nn.sigmoid(g1[:, :H])
        z1 = jax.nn.sigmoid(g1[:, H:2 * H])
        n1 = jnp.tanh(g1[:, 2 * H:3 * H] + r1 * g1[:, 3 * H:])
        h1n = (1.0 - z1) * n1 + z1 * h1
        if i == 0:
            h1 = h1n * (c > 0).astype(jnp.float32)
        else:
            h1 = h1n

    h0_ref[...] = h0
    h1_ref[...] = h1

    @pl.when(c == NCH - 1)
    def _():
        cat1 = jnp.concatenate([h0, h1], axis=1)
        g1 = jnp.dot(_tile8(cat1) * m1, W1s,
                     preferred_element_type=jnp.float32) + b1r
        r1 = jax.nn.sigmoid(g1[:, :H])
        z1 = jax.nn.sigmoid(g1[:, H:2 * H])
        n1 = jnp.tanh(g1[:, 2 * H:3 * H] + r1 * g1[:, 3 * H:])
        h1f = (1.0 - z1) * n1 + z1 * h1

        zh = jnp.dot(_tile8(h1f) * m0, Wh1s_ref[...],
                     preferred_element_type=jnp.float32) + bh1r
        zh_r = jnp.maximum(zh, 0.0)
        pred = jnp.sum(zh_r * wh2r, axis=1, keepdims=True) + bh2r
        ws = ws_ref[...]
        out_ref[...] = (ws[:B] * pred[:B]) + (ws[B:] * pred[B:])


@jax.jit
def kernel(x, horizon, W_in, b_in, emb, W_gate, b_gate, W_ih0, W_hh0, b_ih0,
           b_hh0, W_ih1, W_hh1, b_ih1, b_hh1, W_h1, b_h1, W_h2, b_h2):
    x = x.astype(jnp.float32)
    horizon = horizon.astype(jnp.int32)

    xp, eid, ws = pl.pallas_call(
        _stage1_kernel,
        grid=(B // BT,),
        in_specs=[
            pl.BlockSpec((BT, L, F), lambda i: (i, 0, 0)),
            pl.BlockSpec((BT,), lambda i: (i,)),
            pl.BlockSpec((D, F), lambda i: (0, 0)),
            pl.BlockSpec((D,), lambda i: (0,)),
            pl.BlockSpec((VOCAB, D), lambda i: (0, 0)),
            pl.BlockSpec((E, D), lambda i: (0, 0)),
            pl.BlockSpec((E,), lambda i: (0,)),
        ],
        out_specs=[
            pl.BlockSpec((BT, L, D), lambda i: (i, 0, 0)),
            pl.BlockSpec((2, BT, 1), lambda i: (0, i, 0)),
            pl.BlockSpec((2, BT, 1), lambda i: (0, i, 0)),
        ],
        out_shape=[
            jax.ShapeDtypeStruct((B, L, D), jnp.float32),
            jax.ShapeDtypeStruct((2, B, 1), jnp.int32),
            jax.ShapeDtypeStruct((2, B, 1), jnp.float32),
        ],
    )(x, horizon, W_in, b_in, emb, W_gate, b_gate)

    Wih0T = W_ih0.transpose(0, 2, 1)
    Whh0s = W_hh0.transpose(0, 2, 1).reshape(E * H, 3 * H)
    Wih1T = W_ih1.transpose(0, 2, 1)
    Whh1T = W_hh1.transpose(0, 2, 1)
    zH = jnp.zeros((E, H, H), jnp.float32)
    top = jnp.concatenate([Wih1T[:, :, :2 * H], Wih1T[:, :, 2 * H:], zH], 2)
    bot = jnp.concatenate([Whh1T[:, :, :2 * H], zH, Whh1T[:, :, 2 * H:]], 2)
    W1s = jnp.concatenate([top, bot], axis=1).reshape(E * 2 * H, 4 * H)
    b1 = jnp.concatenate([b_ih1[:, :2 * H] + b_hh1[:, :2 * H],
                          b_ih1[:, 2 * H:], b_hh1[:, 2 * H:]], axis=1)
    Wh1s = W_h1.transpose(0, 2, 1).reshape(E * H, HEAD)

    out = pl.pallas_call(
        _stage2_kernel,
        grid=(NCH,),
        in_specs=[
            pl.BlockSpec((B, TC, D), lambda c: (0, c, 0)),
            pl.BlockSpec((R, 1), lambda c: (0, 0)),
            pl.BlockSpec((R, 1), lambda c: (0, 0)),
            pl.BlockSpec((E, D, 3 * H), lambda c: (0, 0, 0)),
            pl.BlockSpec((E * H, 3 * H), lambda c: (0, 0)),
            pl.BlockSpec((E, 3 * H), lambda c: (0, 0)),
            pl.BlockSpec((E, 3 * H), lambda c: (0, 0)),
            pl.BlockSpec((E * 2 * H, 4 * H), lambda c: (0, 0)),
            pl.BlockSpec((E, 4 * H), lambda c: (0, 0)),
            pl.BlockSpec((E * H, HEAD), lambda c: (0, 0)),
            pl.BlockSpec((E, HEAD), lambda c: (0, 0)),
            pl.BlockSpec((E, HEAD), lambda c: (0, 0)),
            pl.BlockSpec((E, 1), lambda c: (0, 0)),
        ],
        out_specs=pl.BlockSpec((B, 1), lambda c: (0, 0)),
        out_shape=jax.ShapeDtypeStruct((B, 1), jnp.float32),
        scratch_shapes=[pltpu.VMEM((R, H), jnp.float32),
                        pltpu.VMEM((R, H), jnp.float32)],
    )(xp, eid.reshape(R, 1), ws.reshape(R, 1), Wih0T, Whh0s, b_ih0, b_hh0,
      W1s, b1, Wh1s, b_h1, W_h2.reshape(E, HEAD), b_h2)

    return out[:, 0]
